# Initial kernel scaffold; baseline (speedup 1.0000x reference)
import jax
import jax.numpy as jnp
from jax.experimental import pallas as pl


def kernel(x):
    raise NotImplementedError("write your pallas kernel here")


# speedup vs baseline: 4.1890x; 4.1890x over previous
import functools

import jax
import jax.numpy as jnp
from jax import lax
from jax.experimental import pallas as pl
from jax.experimental.pallas import tpu as pltpu
from jax.experimental.pallas import tpu_sc as plsc

VOC = 128
_B, _C, _W = 1024, 26, 20
N = _B * _C * _W
L = 16

_INFO = plsc.get_sparse_core_info()
NC = _INFO.num_cores
NS = _INFO.num_subcores
NW = NC * NS
ROWS_W = N // NW

CHUNK = 320
NCHUNK = ROWS_W // CHUNK
GROUPS = CHUNK // L
CHUNKV = CHUNK * VOC


def _onehot_body(x_hbm, out_hbm, idx_v, buf_v, sem0, sem1):
    wid = lax.axis_index("s") * NC + lax.axis_index("c")
    base = wid * ROWS_W

    pltpu.sync_copy(x_hbm.at[pl.ds(base, ROWS_W)], idx_v)

    iota = lax.iota(jnp.int32, L)
    ones = jnp.full((L,), 1.0, jnp.float32)
    zeros = jnp.zeros((L,), jnp.float32)

    def zinit(i, carry):
        for u in range(8):
            buf_v[pl.ds(i * (8 * L) + u * L, L)] = zeros
        return carry

    lax.fori_loop(0, (2 * CHUNKV) // (8 * L), zinit, 0)

    def scatter_chunk(ci, boff, val):
        local = ci * CHUNK

        def g(gi, carry):
            xv = idx_v[pl.ds(local + gi * L, L)]
            pos = (iota + gi * L) * VOC + xv + boff
            plsc.store_scatter(buf_v, [pos], val)
            return carry

        lax.fori_loop(0, GROUPS, g, 0)

    def dma(ci, boff, sem):
        dst = out_hbm.at[pl.ds((base + ci * CHUNK) * VOC, CHUNKV)]
        src = buf_v.at[pl.ds(boff, CHUNKV)]
        return pltpu.make_async_copy(src, dst, sem)

    scatter_chunk(0, 0, ones)
    dma(0, 0, sem0).start()
    scatter_chunk(1, CHUNKV, ones)
    dma(1, CHUNKV, sem1).start()

    def outer(cc, carry):
        c = cc * 2
        for b, boff, sem in ((0, 0, sem0), (1, CHUNKV, sem1)):
            ci = c + b
            dma(ci - 2, boff, sem).wait()
            scatter_chunk(ci - 2, boff, zeros)
            scatter_chunk(ci, boff, ones)
            dma(ci, boff, sem).start()
        return carry

    lax.fori_loop(1, NCHUNK // 2, outer, 0)

    dma(NCHUNK - 2, 0, sem0).wait()
    dma(NCHUNK - 1, CHUNKV, sem1).wait()


_onehot = functools.partial(
    pl.kernel,
    mesh=plsc.VectorSubcoreMesh(core_axis_name="c", subcore_axis_name="s"),
    compiler_params=pltpu.CompilerParams(needs_layout_passes=False),
    out_type=jax.ShapeDtypeStruct((N * VOC,), jnp.float32),
    scratch_types=[
        pltpu.VMEM((ROWS_W,), jnp.int32),
        pltpu.VMEM((2 * CHUNKV,), jnp.float32),
        pltpu.SemaphoreType.DMA,
        pltpu.SemaphoreType.DMA,
    ],
)(_onehot_body)


@jax.jit
def kernel(x):
    flat = _onehot(x.reshape(N))
    return flat.reshape(_B, _C, _W, VOC)


# speedup vs baseline: 8.0702x; 1.9265x over previous
import functools

import jax
import jax.numpy as jnp
from jax import lax
from jax.experimental import pallas as pl
from jax.experimental.pallas import tpu as pltpu
from jax.experimental.pallas import tpu_sc as plsc

VOC = 128
_B, _C, _W = 1024, 26, 20
N = _B * _C * _W
L = 16

_INFO = plsc.get_sparse_core_info()
NC = _INFO.num_cores
NS = _INFO.num_subcores
NW = NC * NS
B_W = _B // NW
ROWS_W = N // NW

CH_C = 13
CHUNK = CH_C * _W
NCHUNK = 2 * B_W
GROUPS = CHUNK // L
TAIL = CHUNK - GROUPS * L


def _onehot_body(x_hbm, out_hbm, idx_v, buf_a, buf_b, sem0, sem1):
    wid = lax.axis_index("s") * NC + lax.axis_index("c")
    base_row = wid * ROWS_W
    base_b = wid * B_W

    pltpu.sync_copy(x_hbm.at[pl.ds(base_row, ROWS_W)], idx_v)

    iota = lax.iota(jnp.int32, L)
    ones = jnp.full((L,), 1.0, jnp.float32)
    zeros = jnp.zeros((L,), jnp.float32)
    tail_mask = iota < TAIL

    def zinit(i, carry):
        c = i // _W
        w = i % _W
        for u in range(VOC // L):
            buf_a[c, w, pl.ds(u * L, L)] = zeros
            buf_b[c, w, pl.ds(u * L, L)] = zeros
        return carry

    lax.fori_loop(0, CH_C * _W, zinit, 0)

    def scatter_chunk(buf, ci, val):
        local = ci * CHUNK

        def group(g, mask):
            r = g * L + iota
            xv = plsc.load_gather(idx_v, [local + r], mask=mask)
            plsc.store_scatter(buf, [r // _W, r % _W, xv], val, mask=mask)

        def g_body(g, carry):
            group(g, None)
            return carry

        lax.fori_loop(0, GROUPS, g_body, 0)
        group(GROUPS, tail_mask)

    def dma(buf, ci, sem):
        b = base_b + ci // 2
        c0 = (ci % 2) * CH_C
        dst = out_hbm.at[b, pl.ds(c0, CH_C)]
        src = buf.at[:, pl.ds(0, _W)]
        return pltpu.make_async_copy(src, dst, sem)

    scatter_chunk(buf_a, 0, ones)
    dma(buf_a, 0, sem0).start()
    scatter_chunk(buf_b, 1, ones)
    dma(buf_b, 1, sem1).start()

    def outer(cc, carry):
        c = cc * 2
        for buf, b_, sem in ((buf_a, 0, sem0), (buf_b, 1, sem1)):
            ci = c + b_
            dma(buf, ci - 2, sem).wait()
            scatter_chunk(buf, ci - 2, zeros)
            scatter_chunk(buf, ci, ones)
            dma(buf, ci, sem).start()
        return carry

    lax.fori_loop(1, NCHUNK // 2, outer, 0)

    dma(buf_a, NCHUNK - 2, sem0).wait()
    dma(buf_b, NCHUNK - 1, sem1).wait()


_onehot = functools.partial(
    pl.kernel,
    mesh=plsc.VectorSubcoreMesh(core_axis_name="c", subcore_axis_name="s"),
    compiler_params=pltpu.CompilerParams(
        needs_layout_passes=False, use_tc_tiling_on_sc=True
    ),
    out_type=jax.ShapeDtypeStruct((_B, _C, _W, VOC), jnp.float32),
    scratch_types=[
        pltpu.VMEM((ROWS_W,), jnp.int32),
        pltpu.VMEM((CH_C, 24, VOC), jnp.float32),
        pltpu.VMEM((CH_C, 24, VOC), jnp.float32),
        pltpu.SemaphoreType.DMA,
        pltpu.SemaphoreType.DMA,
    ],
)(_onehot_body)


@jax.jit
def kernel(x):
    return _onehot(x.reshape(N))


# speedup vs baseline: 24.3524x; 3.0176x over previous
import functools

import jax
import jax.numpy as jnp
from jax import lax
from jax.experimental import pallas as pl
from jax.experimental.pallas import tpu as pltpu
from jax.experimental.pallas import tpu_sc as plsc

VOC = 128
_B, _C, _W = 1024, 26, 20
N = _B * _C * _W
L = 16

_INFO = plsc.get_sparse_core_info()
NC = _INFO.num_cores
NS = _INFO.num_subcores
NW = NC * NS
ROWS_W = N // NW

CHUNK = 256
NCHUNK = ROWS_W // CHUNK
GROUPS = CHUNK // L
B_CH = _B // CHUNK


def _onehot_body(x_hbm, out_hbm, idx_v, buf_a, buf_b, sem0, sem1):
    wid = lax.axis_index("s") * NC + lax.axis_index("c")
    base_row = wid * ROWS_W
    base_chunk = wid * NCHUNK

    pltpu.sync_copy(x_hbm.at[pl.ds(base_row, ROWS_W)], idx_v)

    iota = lax.iota(jnp.int32, L)
    ones = jnp.full((L,), 1.0, jnp.float32)
    zeros = jnp.zeros((L,), jnp.float32)

    def zinit(i, carry):
        for u in range(VOC // L):
            buf_a[i, pl.ds(u * L, L)] = zeros
            buf_b[i, pl.ds(u * L, L)] = zeros
        return carry

    lax.fori_loop(0, CHUNK, zinit, 0)

    def scatter_chunk(buf, ci, val):
        local = ci * CHUNK

        def g_body(g, carry):
            xv = idx_v[pl.ds(local + g * L, L)]
            plsc.store_scatter(buf, [g * L + iota, xv], val)
            return carry

        lax.fori_loop(0, GROUPS, g_body, 0)

    def dma(buf, ci, sem):
        g = base_chunk + ci
        s = g // B_CH
        b0 = (g % B_CH) * CHUNK
        dst = out_hbm.at[s // _W, s % _W, pl.ds(b0, CHUNK)]
        return pltpu.make_async_copy(buf, dst, sem)

    scatter_chunk(buf_a, 0, ones)
    dma(buf_a, 0, sem0).start()
    scatter_chunk(buf_b, 1, ones)
    dma(buf_b, 1, sem1).start()

    def outer(cc, carry):
        c = cc * 2
        for buf, b_, sem in ((buf_a, 0, sem0), (buf_b, 1, sem1)):
            ci = c + b_
            dma(buf, ci - 2, sem).wait()
            scatter_chunk(buf, ci - 2, zeros)
            scatter_chunk(buf, ci, ones)
            dma(buf, ci, sem).start()
        return carry

    lax.fori_loop(1, (NCHUNK - 1) // 2, outer, 0)

    dma(buf_a, NCHUNK - 3, sem0).wait()
    scatter_chunk(buf_a, NCHUNK - 3, zeros)
    scatter_chunk(buf_a, NCHUNK - 1, ones)
    dma(buf_a, NCHUNK - 1, sem0).start()

    dma(buf_b, NCHUNK - 2, sem1).wait()
    dma(buf_a, NCHUNK - 1, sem0).wait()


_onehot = functools.partial(
    pl.kernel,
    mesh=plsc.VectorSubcoreMesh(core_axis_name="c", subcore_axis_name="s"),
    compiler_params=pltpu.CompilerParams(
        needs_layout_passes=False, use_tc_tiling_on_sc=True
    ),
    out_type=jax.ShapeDtypeStruct((_C, _W, _B, VOC), jnp.float32),
    scratch_types=[
        pltpu.VMEM((ROWS_W,), jnp.int32),
        pltpu.VMEM((CHUNK, VOC), jnp.float32),
        pltpu.VMEM((CHUNK, VOC), jnp.float32),
        pltpu.SemaphoreType.DMA,
        pltpu.SemaphoreType.DMA,
    ],
)(_onehot_body)


@jax.jit
def kernel(x):
    xt = jnp.transpose(x, (1, 2, 0)).reshape(N)
    out = _onehot(xt)
    return jnp.transpose(out, (2, 0, 1, 3))
